# baseline (speedup 1.0000x reference)
import jax
import jax.numpy as jnp
from jax.experimental import pallas as pl
from jax.experimental.pallas import tpu as pltpu

_N = 1024


def _gcn_kernel(pts_ref, ptsT_ref, w1_ref, b1_ref, w2_ref, b2_ref,
                w3_ref, b3_ref, out_ref):
    f32 = jnp.float32
    hi = jax.lax.Precision.DEFAULT
    for s in range(pts_ref.shape[0]):
        px_col = pts_ref[s, :, 0:1]
        py_col = pts_ref[s, :, 1:2]
        px_row = ptsT_ref[s, 0:1, :]
        py_row = ptsT_ref[s, 1:2, :]
        dx = px_col - px_row
        dy = py_col - py_row
        d2 = dx * dx + dy * dy
        a = (d2 < 1.0).astype(f32)
        deg_col = jnp.sum(a, axis=1, keepdims=True) + 1.0
        dinv_col = 1.0 / jnp.sqrt(deg_col)

        def propagate(xw, b_row):
            y = dinv_col * xw
            z = jnp.dot(a, y, preferred_element_type=f32, precision=hi) + y
            return dinv_col * z + b_row

        xw = px_col * w1_ref[0:1, :] + py_col * w1_ref[1:2, :]
        x = jnp.maximum(propagate(xw, b1_ref[0:1, :]), 0.0)
        x = jnp.maximum(propagate(jnp.dot(x, w2_ref[...],
                                          preferred_element_type=f32,
                                          precision=hi), b2_ref[0:1, :]), 0.0)
        out_ref[s, :, :] = propagate(jnp.dot(x, w3_ref[...],
                                             preferred_element_type=f32,
                                             precision=hi), b3_ref[0:1, :])


@jax.jit
def kernel(points, W1, b1, W2, b2, W3, b3):
    B, N, _ = points.shape
    d = W1.shape[1]
    pts = points.astype(jnp.float32)
    ptsT = jnp.transpose(pts, (0, 2, 1))
    SB = 2
    full = lambda s: pl.BlockSpec(s, lambda i: (0,) * len(s))
    grid_spec = pltpu.PrefetchScalarGridSpec(
        num_scalar_prefetch=0,
        grid=(B // SB,),
        in_specs=[
            pl.BlockSpec((SB, N, 2), lambda i: (i, 0, 0)),
            pl.BlockSpec((SB, 2, N), lambda i: (i, 0, 0)),
            full(W1.shape),
            full((1, d)),
            full(W2.shape),
            full((1, 2 * d)),
            full(W3.shape),
            full((1, 4 * d)),
        ],
        out_specs=pl.BlockSpec((SB, N, 4 * d), lambda i: (i, 0, 0)),
    )
    return pl.pallas_call(
        _gcn_kernel,
        grid_spec=grid_spec,
        out_shape=jax.ShapeDtypeStruct((B, N, 4 * d), jnp.float32),
        compiler_params=pltpu.CompilerParams(
            dimension_semantics=("arbitrary",),
        ),
    )(pts, ptsT, W1, b1.reshape(1, d), W2, b2.reshape(1, 2 * d),
      W3, b3.reshape(1, 4 * d))


# speedup vs baseline: 1.0328x; 1.0328x over previous
import jax
import jax.numpy as jnp
from jax.experimental import pallas as pl
from jax.experimental.pallas import tpu as pltpu

_N = 1024


def _gcn_kernel(pts_ref, ptsT_ref, w1_ref, b1_ref, w2_ref, b2_ref,
                w3_ref, b3_ref, out_ref):
    f32 = jnp.float32
    hi = jax.lax.Precision.DEFAULT
    for s in range(pts_ref.shape[0]):
        px_col = pts_ref[s, :, 0:1]
        py_col = pts_ref[s, :, 1:2]
        px_row = ptsT_ref[s, 0:1, :]
        py_row = ptsT_ref[s, 1:2, :]
        dx = px_col - px_row
        dy = py_col - py_row
        d2 = dx * dx + dy * dy
        a = (d2 < 1.0).astype(f32)
        deg_col = jnp.sum(a, axis=1, keepdims=True) + 1.0
        dinv_col = 1.0 / jnp.sqrt(deg_col)

        def propagate(xw, b_row):
            y = dinv_col * xw
            z = jnp.dot(a, y, preferred_element_type=f32, precision=hi) + y
            return dinv_col * z + b_row

        xw = px_col * w1_ref[0:1, :] + py_col * w1_ref[1:2, :]
        x = jnp.maximum(propagate(xw, b1_ref[0:1, :]), 0.0)
        x = jnp.maximum(propagate(jnp.dot(x, w2_ref[...],
                                          preferred_element_type=f32,
                                          precision=hi), b2_ref[0:1, :]), 0.0)
        out_ref[s, :, :] = propagate(jnp.dot(x, w3_ref[...],
                                             preferred_element_type=f32,
                                             precision=hi), b3_ref[0:1, :])


@jax.jit
def kernel(points, W1, b1, W2, b2, W3, b3):
    B, N, _ = points.shape
    d = W1.shape[1]
    pts = points.astype(jnp.float32)
    ptsT = jnp.transpose(pts, (0, 2, 1))
    SB = 4
    full = lambda s: pl.BlockSpec(s, lambda i: (0,) * len(s))
    grid_spec = pltpu.PrefetchScalarGridSpec(
        num_scalar_prefetch=0,
        grid=(B // SB,),
        in_specs=[
            pl.BlockSpec((SB, N, 2), lambda i: (i, 0, 0)),
            pl.BlockSpec((SB, 2, N), lambda i: (i, 0, 0)),
            full(W1.shape),
            full((1, d)),
            full(W2.shape),
            full((1, 2 * d)),
            full(W3.shape),
            full((1, 4 * d)),
        ],
        out_specs=pl.BlockSpec((SB, N, 4 * d), lambda i: (i, 0, 0)),
    )
    return pl.pallas_call(
        _gcn_kernel,
        grid_spec=grid_spec,
        out_shape=jax.ShapeDtypeStruct((B, N, 4 * d), jnp.float32),
        compiler_params=pltpu.CompilerParams(
            dimension_semantics=("arbitrary",),
        ),
    )(pts, ptsT, W1, b1.reshape(1, d), W2, b2.reshape(1, 2 * d),
      W3, b3.reshape(1, 4 * d))


# speedup vs baseline: 1.0780x; 1.0438x over previous
import jax
import jax.numpy as jnp
from jax.experimental import pallas as pl
from jax.experimental.pallas import tpu as pltpu

_N = 1024


def _gcn_kernel(pts_ref, ptsT_ref, w1_ref, b1_ref, w2_ref, b2_ref,
                w3_ref, b3_ref, out_hbm_ref, stage_ref, sem):
    f32 = jnp.float32
    hi = jax.lax.Precision.DEFAULT
    B = pts_ref.shape[0]
    copies = []
    for s in range(B):
        px_col = pts_ref[s, :, 0:1]
        py_col = pts_ref[s, :, 1:2]
        px_row = ptsT_ref[s, 0:1, :]
        py_row = ptsT_ref[s, 1:2, :]
        dx = px_col - px_row
        dy = py_col - py_row
        d2 = dx * dx + dy * dy
        a = (d2 < 1.0).astype(f32)
        deg_col = jnp.sum(a, axis=1, keepdims=True) + 1.0
        dinv_col = 1.0 / jnp.sqrt(deg_col)

        def propagate(xw, b_row):
            y = dinv_col * xw
            z = jnp.dot(a, y, preferred_element_type=f32, precision=hi) + y
            return dinv_col * z + b_row

        xw = px_col * w1_ref[0:1, :] + py_col * w1_ref[1:2, :]
        x = jnp.maximum(propagate(xw, b1_ref[0:1, :]), 0.0)
        x = jnp.maximum(propagate(jnp.dot(x, w2_ref[...],
                                          preferred_element_type=f32,
                                          precision=hi), b2_ref[0:1, :]), 0.0)
        stage_ref[s, :, :] = propagate(jnp.dot(x, w3_ref[...],
                                               preferred_element_type=f32,
                                               precision=hi), b3_ref[0:1, :])
        cp = pltpu.make_async_copy(stage_ref.at[s], out_hbm_ref.at[s],
                                   sem.at[s])
        cp.start()
        copies.append(cp)
    for cp in copies:
        cp.wait()


@jax.jit
def kernel(points, W1, b1, W2, b2, W3, b3):
    B, N, _ = points.shape
    d = W1.shape[1]
    pts = points.astype(jnp.float32)
    ptsT = jnp.transpose(pts, (0, 2, 1))
    full = lambda s: pl.BlockSpec(s, lambda i: (0,) * len(s))
    grid_spec = pltpu.PrefetchScalarGridSpec(
        num_scalar_prefetch=0,
        grid=(1,),
        in_specs=[
            pl.BlockSpec((B, N, 2), lambda i: (0, 0, 0)),
            pl.BlockSpec((B, 2, N), lambda i: (0, 0, 0)),
            full(W1.shape),
            full((1, d)),
            full(W2.shape),
            full((1, 2 * d)),
            full(W3.shape),
            full((1, 4 * d)),
        ],
        out_specs=pl.BlockSpec(memory_space=pl.ANY),
        scratch_shapes=[
            pltpu.VMEM((B, N, 4 * d), jnp.float32),
            pltpu.SemaphoreType.DMA((B,)),
        ],
    )
    return pl.pallas_call(
        _gcn_kernel,
        grid_spec=grid_spec,
        out_shape=jax.ShapeDtypeStruct((B, N, 4 * d), jnp.float32),
    )(pts, ptsT, W1, b1.reshape(1, d), W2, b2.reshape(1, 2 * d),
      W3, b3.reshape(1, 4 * d))


# speedup vs baseline: 1.0809x; 1.0026x over previous
import jax
import jax.numpy as jnp
from jax.experimental import pallas as pl
from jax.experimental.pallas import tpu as pltpu


def _gcn_kernel(pts_ref, ptsT_ref, w1_ref, b1_ref, w2_ref, b2_ref,
                w3_ref, b3_ref, out_hbm_ref, stage_ref, sem):
    f32 = jnp.float32
    hi = jax.lax.Precision.DEFAULT
    B = pts_ref.shape[0]
    copies = []
    for s in range(B):
        px_col = pts_ref[s, :, 0:1]
        py_col = pts_ref[s, :, 1:2]
        px_row = ptsT_ref[s, 0:1, :]
        py_row = ptsT_ref[s, 1:2, :]
        dx = px_col - px_row
        dy = py_col - py_row
        d2 = dx * dx + dy * dy
        a = (d2 < 1.0).astype(f32)
        deg_col = jnp.sum(a, axis=1, keepdims=True) + 1.0
        dinv_col = 1.0 / jnp.sqrt(deg_col)

        def propagate(xw, b_row):
            y = dinv_col * xw
            z = jnp.dot(a, y, preferred_element_type=f32, precision=hi) + y
            return dinv_col * z + b_row

        xw = px_col * w1_ref[0:1, :] + py_col * w1_ref[1:2, :]
        x = jnp.maximum(propagate(xw, b1_ref[0:1, :]), 0.0)
        x = jnp.maximum(propagate(jnp.dot(x, w2_ref[...],
                                          preferred_element_type=f32,
                                          precision=hi), b2_ref[0:1, :]), 0.0)
        stage_ref[s, :, :] = propagate(jnp.dot(x, w3_ref[...],
                                               preferred_element_type=f32,
                                               precision=hi), b3_ref[0:1, :])
        cp = pltpu.make_async_copy(stage_ref.at[s], out_hbm_ref.at[s],
                                   sem.at[s])
        cp.start()
        copies.append(cp)
    for cp in copies:
        cp.wait()


@jax.jit
def kernel(points, W1, b1, W2, b2, W3, b3):
    B, N, _ = points.shape
    d = W1.shape[1]
    pts = points.astype(jnp.float32)
    ptsT = jnp.transpose(pts, (0, 2, 1))
    full = lambda s: pl.BlockSpec(s, lambda i: (0,) * len(s))
    grid_spec = pltpu.PrefetchScalarGridSpec(
        num_scalar_prefetch=0,
        grid=(1,),
        in_specs=[
            full((B, N, 2)),
            full((B, 2, N)),
            full(W1.shape),
            full((1, d)),
            full(W2.shape),
            full((1, 2 * d)),
            full(W3.shape),
            full((1, 4 * d)),
        ],
        out_specs=pl.BlockSpec(memory_space=pl.ANY),
        scratch_shapes=[
            pltpu.VMEM((B, N, 4 * d), jnp.float32),
            pltpu.SemaphoreType.DMA((B,)),
        ],
    )
    return pl.pallas_call(
        _gcn_kernel,
        grid_spec=grid_spec,
        out_shape=jax.ShapeDtypeStruct((B, N, 4 * d), jnp.float32),
    )(pts, ptsT, W1, b1.reshape(1, d), W2, b2.reshape(1, 2 * d),
      W3, b3.reshape(1, 4 * d))
